# baseline (speedup 1.0000x reference)
import functools

import jax
import jax.numpy as jnp
from jax import lax
from jax.experimental import pallas as pl
from jax.experimental.pallas import tpu as pltpu
from jax.experimental.pallas import tpu_sc as plsc

D_MODEL = 768
MAX_LEN = 8192
B = 4
L = 2048

NC = 2
NS = 16
NW = NC * NS
L_PER_TILE = L // NW
LCHUNK = 8
CHUNK = LCHUNK * B
NCHUNK = L_PER_TILE // LCHUNK
ROWS_PER_TILE = L_PER_TILE * B
NBUF = 4
PRE = 2

_MESH = plsc.VectorSubcoreMesh(core_axis_name="c", subcore_axis_name="s")


@functools.partial(
    pl.kernel,
    out_type=jax.ShapeDtypeStruct((L + 1, B, D_MODEL), jnp.float32),
    mesh=_MESH,
    scratch_types=[
        pltpu.VMEM((B, L), jnp.int32),
        pltpu.VMEM((ROWS_PER_TILE,), jnp.int32),
        [pltpu.VMEM((CHUNK, 1, D_MODEL), jnp.float32) for _ in range(NBUF)],
        [pltpu.SemaphoreType.DMA for _ in range(NBUF)],
        [pltpu.SemaphoreType.DMA for _ in range(NBUF)],
        [pltpu.SemaphoreType.DMA for _ in range(NBUF)],
        pltpu.SemaphoreType.DMA,
    ],
    compiler_params=pltpu.CompilerParams(needs_layout_passes=False),
)
def _pe_add(x_hbm, ts_hbm, pe_hbm, out_hbm, ts_v, idx_v, bufs, semx, semg,
            sems, semi):
    wid = lax.axis_index("s") * NC + lax.axis_index("c")
    lbase = 1 + wid * L_PER_TILE

    cp_i = pltpu.async_copy(ts_hbm, ts_v, semi)

    @pl.when(wid == 0)
    def _():
        hdr = bufs[0].reshape(LCHUNK, B, D_MODEL)
        pltpu.sync_copy(x_hbm.at[pl.ds(0, 1)], hdr.at[pl.ds(0, 1)])
        pltpu.sync_copy(hdr.at[pl.ds(0, 1)], out_hbm.at[pl.ds(0, 1)])

    def issue_x(c):
        b = c % NBUF
        return pltpu.async_copy(
            x_hbm.at[pl.ds(lbase + c * LCHUNK, LCHUNK)],
            bufs[b].reshape(LCHUNK, B, D_MODEL), semx[b])

    cp_x = [None] * NCHUNK
    cp_g = [None] * NCHUNK
    cp_s = [None] * NCHUNK
    for c in range(min(PRE, NCHUNK)):
        cp_x[c] = issue_x(c)
    cp_i.wait()
    iota = lax.iota(jnp.int32, 16)
    lb0 = wid * L_PER_TILE
    for i in range(ROWS_PER_TILE // 16):
        j = iota + (16 * i)
        lv = lb0 + lax.shift_right_logical(j, 2)
        bv = lax.bitwise_and(j, 3)
        v = plsc.load_gather(ts_v, [bv, lv])
        idx_v[pl.ds(16 * i, 16)] = jnp.minimum(
            jnp.maximum(v, 0), MAX_LEN - 1)
    for c in range(NCHUNK):
        b = c % NBUF
        cp_x[c].wait()
        cp_g[c] = pltpu.async_copy(
            pe_hbm.at[idx_v.at[pl.ds(c * CHUNK, CHUNK)]],
            bufs[b], semg[b], add=True)
        if c > 0:
            cp_g[c - 1].wait()
            pb = (c - 1) % NBUF
            cp_s[c - 1] = pltpu.async_copy(
                bufs[pb].reshape(LCHUNK, B, D_MODEL),
                out_hbm.at[pl.ds(lbase + (c - 1) * LCHUNK, LCHUNK)], sems[pb])
        nxt = c + PRE
        if nxt < NCHUNK:
            if nxt - NBUF >= 0:
                cp_s[nxt - NBUF].wait()
            cp_x[nxt] = issue_x(nxt)
    last = NCHUNK - 1
    cp_g[last].wait()
    cp_s[last] = pltpu.async_copy(
        bufs[last % NBUF].reshape(LCHUNK, B, D_MODEL),
        out_hbm.at[pl.ds(lbase + last * LCHUNK, LCHUNK)], sems[last % NBUF])
    for c in range(max(0, NCHUNK - NBUF), NCHUNK):
        cp_s[c].wait()


def kernel(x, timestamps, pe):
    return _pe_add(x, timestamps, pe)


# speedup vs baseline: 1.0264x; 1.0264x over previous
import functools

import jax
import jax.numpy as jnp
from jax import lax
from jax.experimental import pallas as pl
from jax.experimental.pallas import tpu as pltpu
from jax.experimental.pallas import tpu_sc as plsc

D_MODEL = 768
MAX_LEN = 8192
B = 4
L = 2048

NC = 2
NS = 16
NW = NC * NS
L_PER_TILE = L // NW
LCHUNK = 8
CHUNK = LCHUNK * B
NCHUNK = L_PER_TILE // LCHUNK
ROWS_PER_TILE = L_PER_TILE * B
NBUF = 4
PRE = 2

_MESH = plsc.VectorSubcoreMesh(core_axis_name="c", subcore_axis_name="s")


@functools.partial(
    pl.kernel,
    out_type=jax.ShapeDtypeStruct((L + 1, B, D_MODEL), jnp.float32),
    mesh=_MESH,
    scratch_types=[
        pltpu.VMEM((ROWS_PER_TILE,), jnp.int32),
        [pltpu.VMEM((CHUNK, 1, D_MODEL), jnp.float32) for _ in range(NBUF)],
        [pltpu.SemaphoreType.DMA for _ in range(NBUF)],
        [pltpu.SemaphoreType.DMA for _ in range(NBUF)],
        [pltpu.SemaphoreType.DMA for _ in range(NBUF)],
        pltpu.SemaphoreType.DMA,
    ],
)
def _pe_add(x_hbm, ts_hbm, pe_hbm, out_hbm, idx_v, bufs, semx, semg, sems,
            semi):
    wid = lax.axis_index("s") * NC + lax.axis_index("c")
    lbase = 1 + wid * L_PER_TILE

    cp_i = pltpu.async_copy(
        ts_hbm.at[pl.ds(wid * ROWS_PER_TILE, ROWS_PER_TILE)], idx_v, semi)

    @pl.when(wid == 0)
    def _():
        hdr = bufs[0].reshape(LCHUNK, B, D_MODEL)
        pltpu.sync_copy(x_hbm.at[pl.ds(0, 1)], hdr.at[pl.ds(0, 1)])
        pltpu.sync_copy(hdr.at[pl.ds(0, 1)], out_hbm.at[pl.ds(0, 1)])

    def issue_x(c):
        b = c % NBUF
        return pltpu.async_copy(
            x_hbm.at[pl.ds(lbase + c * LCHUNK, LCHUNK)],
            bufs[b].reshape(LCHUNK, B, D_MODEL), semx[b])

    cp_x = [None] * NCHUNK
    cp_g = [None] * NCHUNK
    cp_s = [None] * NCHUNK
    for c in range(min(PRE, NCHUNK)):
        cp_x[c] = issue_x(c)
    cp_i.wait()
    def clamp_body(i, carry):
        sl = pl.ds(i * 16, 16)
        idx_v[sl] = jnp.minimum(jnp.maximum(idx_v[sl], 0), MAX_LEN - 1)
        return carry
    lax.fori_loop(0, ROWS_PER_TILE // 16, clamp_body, 0)
    for c in range(NCHUNK):
        b = c % NBUF
        cp_x[c].wait()
        cp_g[c] = pltpu.async_copy(
            pe_hbm.at[idx_v.at[pl.ds(c * CHUNK, CHUNK)]],
            bufs[b], semg[b], add=True)
        if c > 0:
            cp_g[c - 1].wait()
            pb = (c - 1) % NBUF
            cp_s[c - 1] = pltpu.async_copy(
                bufs[pb].reshape(LCHUNK, B, D_MODEL),
                out_hbm.at[pl.ds(lbase + (c - 1) * LCHUNK, LCHUNK)], sems[pb])
        nxt = c + PRE
        if nxt < NCHUNK:
            if nxt - NBUF >= 0:
                cp_s[nxt - NBUF].wait()
            cp_x[nxt] = issue_x(nxt)
    last = NCHUNK - 1
    cp_g[last].wait()
    cp_s[last] = pltpu.async_copy(
        bufs[last % NBUF].reshape(LCHUNK, B, D_MODEL),
        out_hbm.at[pl.ds(lbase + last * LCHUNK, LCHUNK)], sems[last % NBUF])
    for c in range(max(0, NCHUNK - NBUF), NCHUNK):
        cp_s[c].wait()


def kernel(x, timestamps, pe):
    ts_flat = timestamps.T.reshape(L * B).astype(jnp.int32)
    return _pe_add(x, ts_flat, pe)
